# baseline (speedup 1.0000x reference)
import functools
import math

import jax
import jax.numpy as jnp
import numpy as np
from jax import lax
from jax.experimental import pallas as pl
from jax.experimental.pallas import tpu as pltpu
from jax.experimental.pallas import tpu_sc as plsc

_PE_LEN = 2048
_LANES = 16
_NC, _NS = 2, 16
_NW = _NC * _NS
_MAX_IDX = 128


def _pos_encoding(length: int, depth: int) -> np.ndarray:
    pos = np.arange(length, dtype=np.float64)[:, None]
    i = np.arange(depth, dtype=np.float64)[None, :]
    exponent = (i - (i % 2)) / depth
    angle = pos / np.power(10000.0, exponent)
    pe = np.where((np.arange(depth)[None, :] % 2) == 0, np.sin(angle), np.cos(angle))
    return np.asarray(pe, dtype=np.float32)


_PE = _pos_encoding(_PE_LEN, 64)


@functools.cache
def _build(B: int, T: int, V: int, D: int):
    assert B % _NW == 0, (B, _NW)
    spw = B // _NW
    scale = np.float32(math.sqrt(D))
    chunks = []
    off = 0
    while off < T:
        n = min(_MAX_IDX, T - off)
        chunks.append((off, n))
        off += n

    mesh = plsc.VectorSubcoreMesh(
        core_axis_name="c", subcore_axis_name="s",
        num_cores=_NC, num_subcores=_NS)

    scratch = []
    for off, n in chunks:
        scratch.append(pltpu.VMEM((n,), jnp.int32))
        scratch.append(pltpu.VMEM((n, D), jnp.float32))
    scratch.append(pltpu.VMEM((T, D), jnp.float32))
    scratch.append(pltpu.SemaphoreType.DMA)

    @functools.partial(
        pl.kernel,
        out_type=jax.ShapeDtypeStruct((B, T, D), jnp.float32),
        mesh=mesh,
        scratch_types=scratch,
        compiler_params=pltpu.CompilerParams(use_tc_tiling_on_sc=False),
    )
    def run(x_hbm, pe_hbm, table_hbm, out_hbm, *refs):
        bufs = []
        for ci in range(len(chunks)):
            bufs.append((refs[2 * ci], refs[2 * ci + 1]))
        pe_v = refs[-2]
        sem = refs[-1]

        wid = lax.axis_index("s") * _NC + lax.axis_index("c")
        base = wid * spw
        pltpu.sync_copy(pe_hbm, pe_v)

        @pl.loop(0, spw)
        def _seq(s):
            seq = base + s
            for (off, n), (idx_v, rows_v) in zip(chunks, bufs):
                pltpu.sync_copy(x_hbm.at[seq, pl.ds(off, n)], idx_v)
                pltpu.async_copy(table_hbm.at[idx_v], rows_v, sem).wait()

                @pl.loop(0, n)
                def _row(r):
                    for k in range(D // _LANES):
                        sl = pl.ds(k * _LANES, _LANES)
                        rows_v[r, sl] = rows_v[r, sl] * scale + pe_v[off + r, sl]

                pltpu.sync_copy(rows_v, out_hbm.at[seq, pl.ds(off, n)])

    return run


def kernel(x, table):
    B, T = x.shape
    V, D = table.shape
    x32 = x.astype(jnp.int32)
    pe = jnp.asarray(_PE[:T, :D])
    return _build(B, T, V, D)(x32, pe, table)


# speedup vs baseline: 1.3368x; 1.3368x over previous
import functools
import math

import jax
import jax.numpy as jnp
import numpy as np
from jax import lax
from jax.experimental import pallas as pl
from jax.experimental.pallas import tpu as pltpu
from jax.experimental.pallas import tpu_sc as plsc

_PE_LEN = 2048
_LANES = 16
_NC, _NS = 2, 16
_NW = _NC * _NS
_MAX_IDX = 128
_NB = 4
_AHEAD = 2


def _pos_encoding(length: int, depth: int) -> np.ndarray:
    pos = np.arange(length, dtype=np.float64)[:, None]
    i = np.arange(depth, dtype=np.float64)[None, :]
    exponent = (i - (i % 2)) / depth
    angle = pos / np.power(10000.0, exponent)
    pe = np.where((np.arange(depth)[None, :] % 2) == 0, np.sin(angle), np.cos(angle))
    return np.asarray(pe, dtype=np.float32)


_PE = _pos_encoding(_PE_LEN, 64)


@functools.cache
def _build(B: int, T: int, V: int, D: int):
    assert B % _NW == 0, (B, _NW)
    spw = B // _NW
    assert spw % _NB == 0 and spw > _NB
    scale = np.float32(math.sqrt(D))
    na = min(_MAX_IDX, T)
    nb = T - na
    assert 0 < nb <= _MAX_IDX and na % 8 == 0

    mesh = plsc.VectorSubcoreMesh(
        core_axis_name="c", subcore_axis_name="s",
        num_cores=_NC, num_subcores=_NS)

    scratch = [
        pltpu.VMEM((spw, T), jnp.int32),
        pltpu.VMEM((T, D), jnp.float32),
    ]
    scratch += [pltpu.VMEM((na, D), jnp.float32) for _ in range(_NB)]
    scratch += [pltpu.VMEM((nb, D), jnp.float32) for _ in range(_NB)]
    scratch += [pltpu.SemaphoreType.DMA for _ in range(2 * _NB)]

    @functools.partial(
        pl.kernel,
        out_type=jax.ShapeDtypeStruct((B, T, D), jnp.float32),
        mesh=mesh,
        scratch_types=scratch,
        compiler_params=pltpu.CompilerParams(use_tc_tiling_on_sc=False),
    )
    def run(x_hbm, pe_hbm, table_hbm, out_hbm, *refs):
        idx_all = refs[0]
        pe_v = refs[1]
        rows_a = refs[2:2 + _NB]
        rows_b = refs[2 + _NB:2 + 2 * _NB]
        gsem = refs[2 + 2 * _NB:2 + 3 * _NB]
        osem = refs[2 + 3 * _NB:2 + 4 * _NB]

        wid = lax.axis_index("s") * _NC + lax.axis_index("c")
        base = wid * spw
        pltpu.sync_copy(pe_hbm, pe_v)
        pltpu.sync_copy(x_hbm.at[pl.ds(base, spw)], idx_all)

        def start_gathers(seq, slot):
            pltpu.async_copy(
                table_hbm.at[idx_all.at[seq, pl.ds(0, na)]],
                rows_a[slot], gsem[slot])
            pltpu.async_copy(
                table_hbm.at[idx_all.at[seq, pl.ds(na, nb)]],
                rows_b[slot], gsem[slot])

        def drain(dst_ref, sem):
            pltpu.make_async_copy(
                out_hbm.at[0, pl.ds(0, dst_ref.shape[0])], dst_ref, sem).wait()

        for i in range(_AHEAD):
            start_gathers(i, i)

        @pl.loop(0, spw, step=_NB)
        def _round(s):
            for b in range(_NB):
                i = s + b
                j = i + _AHEAD
                slot_j = (b + _AHEAD) % _NB

                @pl.when(j < spw)
                def _():
                    @pl.when(j >= _NB)
                    def _():
                        drain(rows_a[slot_j], osem[slot_j])
                        drain(rows_b[slot_j], osem[slot_j])
                    start_gathers(j, slot_j)

                drain(rows_a[b], gsem[b])
                drain(rows_b[b], gsem[b])

                @pl.loop(0, na)
                def _row_a(r):
                    for k in range(D // _LANES):
                        sl = pl.ds(k * _LANES, _LANES)
                        rows_a[b][r, sl] = rows_a[b][r, sl] * scale + pe_v[r, sl]

                @pl.loop(0, nb)
                def _row_b(r):
                    for k in range(D // _LANES):
                        sl = pl.ds(k * _LANES, _LANES)
                        rows_b[b][r, sl] = (rows_b[b][r, sl] * scale
                                            + pe_v[na + r, sl])

                seq = base + i
                pltpu.async_copy(rows_a[b], out_hbm.at[seq, pl.ds(0, na)],
                                 osem[b])
                pltpu.async_copy(rows_b[b], out_hbm.at[seq, pl.ds(na, nb)],
                                 osem[b])

        for b in range(_NB):
            drain(rows_a[b], osem[b])
            drain(rows_b[b], osem[b])

    return run


def kernel(x, table):
    B, T = x.shape
    V, D = table.shape
    x32 = x.astype(jnp.int32)
    pe = jnp.asarray(_PE[:T, :D])
    return _build(B, T, V, D)(x32, pe, table)
